# Initial kernel scaffold; baseline (speedup 1.0000x reference)
import jax
import jax.numpy as jnp
from jax.experimental import pallas as pl


def kernel(x, edge_index, edge_attr, W1, b1, gamma1, beta1, W3, b3):
    raise NotImplementedError("write your pallas kernel here")


# speedup vs baseline: 24.1829x; 24.1829x over previous
import functools

import jax
import jax.numpy as jnp
from jax import lax
from jax.experimental import pallas as pl
from jax.experimental.pallas import tpu as pltpu
from jax.experimental.pallas import tpu_sc as plsc

N = 100000
E = 1600000
EPS = 1e-5

NPAD = 100096

NC, NS, L = 2, 16, 16
NW = NC * NS
CH = 128
NCHUNK = E // CH
RPW = NCHUNK // NW
EXTRA_START = RPW * NW
EXTRA = NCHUNK - EXTRA_START
NPT = NPAD // NS

_mesh = functools.partial(plsc.VectorSubcoreMesh,
                          core_axis_name="c", subcore_axis_name="s")


def _ids():
    cid = lax.axis_index("c")
    sid = lax.axis_index("s")
    return cid, sid, sid * NC + cid


def _sc_deg(col1, w1, zn):
    @functools.partial(
        pl.kernel,
        out_type=jax.ShapeDtypeStruct((NC * NPAD,), jnp.float32),
        mesh=_mesh(),
        scratch_types=[
            pltpu.VMEM_SHARED((NPAD,), jnp.float32),
            pltpu.VMEM((CH,), jnp.int32),
            pltpu.VMEM((CH,), jnp.float32),
            pltpu.VMEM((NPT,), jnp.float32),
        ],
    )
    def k(col_hbm, w_hbm, z_hbm, out_hbm, acc, colb, wb, bb):
        cid, sid, wid = _ids()
        sl = pl.ds(sid * NPT, NPT)
        pltpu.sync_copy(z_hbm.at[sl], bb)
        pltpu.sync_copy(bb, acc.at[sl])
        plsc.subcore_barrier()

        def push(r):
            e0 = r * CH
            pltpu.sync_copy(col_hbm.at[pl.ds(e0, CH)], colb)
            pltpu.sync_copy(w_hbm.at[pl.ds(e0, CH)], wb)
            pltpu.sync_copy(wb, acc.at[colb], add=True)

        start = wid * RPW + jnp.minimum(wid, EXTRA)
        nmine = RPW + jnp.where(wid < EXTRA, 1, 0)

        def body(i, c):
            push(start + i)
            return c

        lax.fori_loop(0, nmine, body, 0)

        plsc.subcore_barrier()
        pltpu.sync_copy(acc.at[sl], bb)
        pltpu.sync_copy(bb, out_hbm.at[pl.ds(cid * NPAD + sid * NPT, NPT)])

    return k(col1, w1, zn)


def _sc_push3(row1, col1, w1, uflat, zn):
    @functools.partial(
        pl.kernel,
        out_type=jax.ShapeDtypeStruct((NC * 3 * NPAD,), jnp.float32),
        mesh=_mesh(),
        scratch_types=[
            pltpu.VMEM_SHARED((NPAD,), jnp.float32),
            pltpu.VMEM_SHARED((NPAD,), jnp.float32),
            pltpu.VMEM_SHARED((NPAD,), jnp.float32),
            pltpu.VMEM_SHARED((NPAD,), jnp.float32),
            pltpu.VMEM_SHARED((NPAD,), jnp.float32),
            pltpu.VMEM_SHARED((NPAD,), jnp.float32),
            pltpu.VMEM((CH,), jnp.int32),
            pltpu.VMEM((CH,), jnp.int32),
            pltpu.VMEM((CH,), jnp.float32),
            pltpu.VMEM((CH,), jnp.float32),
            pltpu.VMEM((CH,), jnp.float32),
            pltpu.VMEM((CH,), jnp.float32),
            pltpu.VMEM((NPT,), jnp.float32),
        ],
    )
    def k(row_hbm, col_hbm, w_hbm, u_hbm, z_hbm, out_hbm,
          u0, u1, u2, p0, p1, p2, rowb, colb, wb, g0, g1, g2, bb):
        cid, sid, wid = _ids()
        sl = pl.ds(sid * NPT, NPT)
        for t, srcc in ((u0, 0), (u1, 1), (u2, 2)):
            pltpu.sync_copy(u_hbm.at[pl.ds(srcc * NPAD + sid * NPT, NPT)], bb)
            pltpu.sync_copy(bb, t.at[sl])
        pltpu.sync_copy(z_hbm.at[sl], bb)
        for t in (p0, p1, p2):
            pltpu.sync_copy(bb, t.at[sl])
        plsc.subcore_barrier()

        def push(r):
            e0 = r * CH
            pltpu.sync_copy(row_hbm.at[pl.ds(e0, CH)], rowb)
            pltpu.sync_copy(col_hbm.at[pl.ds(e0, CH)], colb)
            pltpu.sync_copy(w_hbm.at[pl.ds(e0, CH)], wb)
            pltpu.sync_copy(u0.at[rowb], g0)
            pltpu.sync_copy(u1.at[rowb], g1)
            pltpu.sync_copy(u2.at[rowb], g2)
            for v in range(CH // L):
                vs = pl.ds(v * L, L)
                wv = wb[vs]
                g0[vs] = g0[vs] * wv
                g1[vs] = g1[vs] * wv
                g2[vs] = g2[vs] * wv
            pltpu.sync_copy(g0, p0.at[colb], add=True)
            pltpu.sync_copy(g1, p1.at[colb], add=True)
            pltpu.sync_copy(g2, p2.at[colb], add=True)

        start = wid * RPW + jnp.minimum(wid, EXTRA)
        nmine = RPW + jnp.where(wid < EXTRA, 1, 0)

        def body(i, c):
            push(start + i)
            return c

        lax.fori_loop(0, nmine, body, 0)

        plsc.subcore_barrier()
        for t, dstc in ((p0, 0), (p1, 1), (p2, 2)):
            pltpu.sync_copy(t.at[sl], bb)
            pltpu.sync_copy(
                bb,
                out_hbm.at[pl.ds((cid * 3 + dstc) * NPAD + sid * NPT, NPT)])

    return k(row1, col1, w1, uflat, zn)


def _sc_push1(row1, col1, w1, uz, zn):
    @functools.partial(
        pl.kernel,
        out_type=jax.ShapeDtypeStruct((NC * NPAD,), jnp.float32),
        mesh=_mesh(),
        scratch_types=[
            pltpu.VMEM_SHARED((NPAD,), jnp.float32),
            pltpu.VMEM_SHARED((NPAD,), jnp.float32),
            pltpu.VMEM((CH,), jnp.int32),
            pltpu.VMEM((CH,), jnp.int32),
            pltpu.VMEM((CH,), jnp.float32),
            pltpu.VMEM((CH,), jnp.float32),
            pltpu.VMEM((NPT,), jnp.float32),
        ],
    )
    def k(row_hbm, col_hbm, w_hbm, uz_hbm, z_hbm, out_hbm,
          tz, acc, rowb, colb, wb, gb, bb):
        cid, sid, wid = _ids()
        sl = pl.ds(sid * NPT, NPT)
        pltpu.sync_copy(uz_hbm.at[sl], bb)
        pltpu.sync_copy(bb, tz.at[sl])
        pltpu.sync_copy(z_hbm.at[sl], bb)
        pltpu.sync_copy(bb, acc.at[sl])
        plsc.subcore_barrier()

        def push(r):
            e0 = r * CH
            pltpu.sync_copy(row_hbm.at[pl.ds(e0, CH)], rowb)
            pltpu.sync_copy(col_hbm.at[pl.ds(e0, CH)], colb)
            pltpu.sync_copy(w_hbm.at[pl.ds(e0, CH)], wb)
            pltpu.sync_copy(tz.at[rowb], gb)
            for v in range(CH // L):
                vs = pl.ds(v * L, L)
                gb[vs] = gb[vs] * wb[vs]
            pltpu.sync_copy(gb, acc.at[colb], add=True)

        start = wid * RPW + jnp.minimum(wid, EXTRA)
        nmine = RPW + jnp.where(wid < EXTRA, 1, 0)

        def body(i, c):
            push(start + i)
            return c

        lax.fori_loop(0, nmine, body, 0)

        plsc.subcore_barrier()
        pltpu.sync_copy(acc.at[sl], bb)
        pltpu.sync_copy(bb, out_hbm.at[pl.ds(cid * NPAD + sid * NPT, NPT)])

    return k(row1, col1, w1, uz, zn)


def _tc_prep(degp, xT):
    def body(degp_ref, xT_ref, dinv_ref, u_ref):
        deg = degp_ref[0, :] + degp_ref[1, :] + 1.0
        dinv = jnp.where(deg > 0, lax.rsqrt(deg), 0.0)
        dinv_ref[0, :] = dinv
        u_ref[...] = xT_ref[...] * dinv[None, :]

    return pl.pallas_call(
        body,
        out_shape=(jax.ShapeDtypeStruct((1, NPAD), jnp.float32),
                   jax.ShapeDtypeStruct((3, NPAD), jnp.float32)),
    )(degp, xT)


def _tc_mid(pp, uT, dinv, W1, g1, be1, W3):
    def body(pp_ref, u_ref, dinv_ref, W1_ref, g1_ref, be1_ref, W3_ref,
             uz_ref):
        dv = dinv_ref[0, :]
        y = (pp_ref[0] + pp_ref[1] + u_ref[...]) * dv[None, :]
        yr = [y[i, :] for i in range(3)]
        mbar = [jnp.sum(yr[i]) / N for i in range(3)]
        S = [[jnp.sum(yr[i] * yr[j]) / N for j in range(3)] for i in range(3)]
        W1m = W1_ref[...]
        Wr = [W1m[i, :] for i in range(3)]
        mh = Wr[0] * mbar[0] + Wr[1] * mbar[1] + Wr[2] * mbar[2]
        q = jnp.zeros_like(mh)
        for i in range(3):
            for j in range(3):
                q = q + Wr[i] * Wr[j] * S[i][j]
        varh = q - mh * mh
        scale = g1_ref[...] * lax.rsqrt(varh + EPS)
        sh = be1_ref[...] - mh * scale
        w3 = W3_ref[...][:, 0]
        acc = jnp.zeros_like(dv)
        for c in range(32):
            tc = (Wr[0][c] * yr[0] + Wr[1][c] * yr[1] + Wr[2][c] * yr[2])
            a = jnp.maximum(tc * scale[c] + sh[c], 0.0)
            acc = acc + w3[c] * a
        uz_ref[...] = (acc * dv)[None, :]

    return pl.pallas_call(
        body,
        out_shape=jax.ShapeDtypeStruct((1, NPAD), jnp.float32),
    )(pp, uT, dinv, W1, g1, be1, W3)


def _tc_final(op, uz, dinv, b3):
    def body(op_ref, uz_ref, dinv_ref, b3_ref, out_ref):
        s = op_ref[0] + op_ref[1] + uz_ref[0, :]
        out_ref[...] = (s * dinv_ref[0, :])[None, :] + b3_ref[...][:, None]

    return pl.pallas_call(
        body,
        out_shape=jax.ShapeDtypeStruct((1, NPAD), jnp.float32),
    )(op, uz, dinv, b3)


def kernel(x, edge_index, edge_attr, W1, b1, gamma1, beta1, W3, b3):
    row1 = edge_index[0]
    col1 = edge_index[1]
    w1 = edge_attr
    zn = jnp.zeros((NPAD,), jnp.float32)
    xT = jnp.pad(x.T, ((0, 0), (0, NPAD - N)))

    degp = _sc_deg(col1, w1, zn).reshape(NC, NPAD)
    dinv, uT = _tc_prep(degp, xT)
    pp = _sc_push3(row1, col1, w1, uT.reshape(3 * NPAD), zn)
    pp = pp.reshape(NC, 3, NPAD)
    uz = _tc_mid(pp, uT, dinv, W1, gamma1, beta1, W3)
    op = _sc_push1(row1, col1, w1, uz.reshape(NPAD), zn).reshape(NC, NPAD)
    outf = _tc_final(op, uz, dinv, b3)
    return outf[0, :N].reshape(N, 1)
